# Initial kernel scaffold; baseline (speedup 1.0000x reference)
import jax
import jax.numpy as jnp
from jax.experimental import pallas as pl


def kernel(x, edge_index, W):
    raise NotImplementedError("write your pallas kernel here")


# speedup vs baseline: 5.4414x; 5.4414x over previous
import functools

import jax
import jax.numpy as jnp
from jax import lax
from jax.experimental import pallas as pl
from jax.experimental.pallas import tpu as pltpu
from jax.experimental.pallas import tpu_sc as plsc

_N_NODES = 10000
_N_PAD = 10240
_N_EDGES = 320000
_DIM = 128
_NC = 2
_NS = 16
_NW = _NC * _NS
_EPW = _N_EDGES // _NW
_CHUNK = 80
_NCHUNK = _EPW // _CHUNK
_RPT = _N_PAD // _NS
_ZR = 128


def _sc_aggregate(x, src, dst):
    mesh = plsc.VectorSubcoreMesh(core_axis_name="c", subcore_axis_name="s")

    @functools.partial(
        pl.kernel,
        out_type=jax.ShapeDtypeStruct((_NC, _N_PAD, _DIM), jnp.float32),
        mesh=mesh,
        scratch_types=[
            pltpu.VMEM_SHARED((_N_PAD, _DIM), jnp.float32),
            pltpu.VMEM((_CHUNK,), jnp.int32),
            pltpu.VMEM((_CHUNK,), jnp.int32),
            pltpu.VMEM((_CHUNK, _DIM), jnp.float32),
            pltpu.VMEM((_ZR, _DIM), jnp.float32),
            pltpu.SemaphoreType.DMA,
        ],
    )
    def k(x_hbm, src_hbm, dst_hbm, out_hbm, accum, src_v, dst_v, rows_v,
          stage_v, sem):
        c = lax.axis_index("c")
        s = lax.axis_index("s")
        w = s * _NC + c

        def zero_row(r, carry):
            for j in range(_DIM // 16):
                stage_v[r, pl.ds(j * 16, 16)] = jnp.zeros((16,), jnp.float32)
            return carry

        lax.fori_loop(0, _ZR, zero_row, 0)
        row0 = s * _RPT
        for j in range(_RPT // _ZR):
            pltpu.sync_copy(stage_v, accum.at[pl.ds(row0 + j * _ZR, _ZR)])
        plsc.subcore_barrier()

        base = w * _EPW

        def edge_chunk(i, carry):
            off = base + i * _CHUNK
            pltpu.sync_copy(src_hbm.at[pl.ds(off, _CHUNK)], src_v)
            pltpu.sync_copy(dst_hbm.at[pl.ds(off, _CHUNK)], dst_v)
            pltpu.async_copy(x_hbm.at[src_v], rows_v, sem).wait()
            pltpu.sync_copy(rows_v, accum.at[dst_v], add=True)
            return carry

        lax.fori_loop(0, _NCHUNK, edge_chunk, 0)
        plsc.subcore_barrier()

        for j in range(_RPT // _ZR):
            r = row0 + j * _ZR
            pltpu.sync_copy(accum.at[pl.ds(r, _ZR)], stage_v)
            pltpu.sync_copy(stage_v, out_hbm.at[c].at[pl.ds(r, _ZR)])

    return k(x, src, dst)


def _mm_relu(partials, W):
    blk = 1024

    def body(p0_ref, p1_ref, w_ref, o_ref):
        ssum = p0_ref[...] + p1_ref[...]
        o_ref[...] = jnp.maximum(
            jnp.dot(ssum, w_ref[...], preferred_element_type=jnp.float32),
            0.0)

    return pl.pallas_call(
        body,
        grid=(_N_PAD // blk,),
        in_specs=[
            pl.BlockSpec((blk, _DIM), lambda i: (i, 0)),
            pl.BlockSpec((blk, _DIM), lambda i: (i, 0)),
            pl.BlockSpec((_DIM, _DIM), lambda i: (0, 0)),
        ],
        out_specs=pl.BlockSpec((blk, _DIM), lambda i: (i, 0)),
        out_shape=jax.ShapeDtypeStruct((_N_NODES, _DIM), jnp.float32),
    )(partials[0], partials[1], W)


def kernel(x, edge_index, W):
    src = edge_index[1].astype(jnp.int32)
    dst = edge_index[0].astype(jnp.int32)
    partials = _sc_aggregate(x, src, dst)
    return _mm_relu(partials, W)
